# baseline (speedup 1.0000x reference)
import functools

import jax
import jax.numpy as jnp
from jax import lax
from jax.experimental import pallas as pl
from jax.experimental.pallas import tpu as pltpu
from jax.experimental.pallas import tpu_sc as plsc

B, S, D = 1, 2048, 768
FFN = 3072
E = 8
TOPK = 2
NPAIR = S * TOPK
TTR = 128
NPAD = NPAIR + E * TTR
NTILES = NPAD // TTR
DI = D // 2

_sc_info = plsc.get_sparse_core_info()
_NC = _sc_info.num_cores
NW = _NC * _sc_info.num_subcores
RPW = NPAD // NW
TPW = S // NW
NCH = D // 16


def _logits_body(hs_ref, gate_ref, logits_ref):
    logits_f = jax.lax.dot_general(
        hs_ref[...], gate_ref[...], (((1,), (1,)), ((), ())),
        preferred_element_type=jnp.float32)
    logits_ref[...] = logits_f.astype(jnp.bfloat16)


def _logits(hs2d, gate_w):
    return pl.pallas_call(
        _logits_body,
        out_shape=jax.ShapeDtypeStruct((S, E), jnp.bfloat16),
    )(hs2d, gate_w)


@functools.partial(
    pl.kernel,
    mesh=plsc.VectorSubcoreMesh(core_axis_name="c", subcore_axis_name="s"),
    out_type=jax.ShapeDtypeStruct((NPAD, DI), jnp.int32),
    scratch_types=[
        pltpu.VMEM((RPW,), jnp.int32),
        pltpu.VMEM((4, RPW // 4, DI), jnp.int32),
        pltpu.SemaphoreType.DMA,
        pltpu.SemaphoreType.DMA,
        pltpu.SemaphoreType.DMA,
        pltpu.SemaphoreType.DMA,
        pltpu.SemaphoreType.DMA,
        pltpu.SemaphoreType.DMA,
        pltpu.SemaphoreType.DMA,
        pltpu.SemaphoreType.DMA,
    ],
)
def _sc_dispatch(hs_hbm, idx_hbm, out_hbm, idx_v, rows_v,
                 g0, g1, g2, g3, w0, w1, w2, w3):
    wid = lax.axis_index("s") * _NC + lax.axis_index("c")
    base = wid * RPW
    ch = RPW // 4
    gsems = (g0, g1, g2, g3)
    wsems = (w0, w1, w2, w3)
    pltpu.sync_copy(idx_hbm.at[pl.ds(base, RPW)], idx_v)
    gcopies = []
    for b in range(4):
        gcopies.append(pltpu.async_copy(
            hs_hbm.at[idx_v.at[pl.ds(b * ch, ch)]], rows_v.at[b], gsems[b]))
    wcopies = []
    for b in range(4):
        gcopies[b].wait()
        wcopies.append(pltpu.async_copy(
            rows_v.at[b], out_hbm.at[pl.ds(base + b * ch, ch)], wsems[b]))
    for b in range(4):
        wcopies[b].wait()


def _ffn_body(meta_ref, x_ref, w_ref, wg_ref, wu_ref, wd_ref, y_ref):
    j = pl.program_id(0)

    @pl.when(j < meta_ref[NTILES])
    def _compute():
        x = x_ref[...]
        g = jax.lax.dot_general(x, wg_ref[0], (((1,), (1,)), ((), ())),
                                preferred_element_type=jnp.float32)
        u = jax.lax.dot_general(x, wu_ref[0], (((1,), (1,)), ((), ())),
                                preferred_element_type=jnp.float32)
        h = (g * jax.nn.sigmoid(g) * u).astype(jnp.bfloat16)
        d = jax.lax.dot_general(h, wd_ref[0], (((1,), (1,)), ((), ())),
                                preferred_element_type=jnp.float32)
        y_ref[...] = d.astype(jnp.bfloat16).astype(jnp.float32) * w_ref[...]


def _ffn(meta, x_sorted, w_sorted, w_gate, w_up, w_down):
    grid_spec = pltpu.PrefetchScalarGridSpec(
        num_scalar_prefetch=1,
        grid=(NTILES,),
        in_specs=[
            pl.BlockSpec((TTR, D), lambda j, m: (j, 0)),
            pl.BlockSpec((TTR, 1), lambda j, m: (j, 0)),
            pl.BlockSpec((1, FFN, D), lambda j, m: (m[j], 0, 0)),
            pl.BlockSpec((1, FFN, D), lambda j, m: (m[j], 0, 0)),
            pl.BlockSpec((1, D, FFN), lambda j, m: (m[j], 0, 0)),
        ],
        out_specs=pl.BlockSpec((TTR, D), lambda j, m: (j, 0)),
    )
    return pl.pallas_call(
        _ffn_body,
        grid_spec=grid_spec,
        out_shape=jax.ShapeDtypeStruct((NPAD, D), jnp.float32),
    )(meta, x_sorted, w_sorted, w_gate, w_up, w_down)


@functools.partial(
    pl.kernel,
    mesh=plsc.VectorSubcoreMesh(core_axis_name="c", subcore_axis_name="s"),
    out_type=jax.ShapeDtypeStruct((S, D), jnp.float32),
    scratch_types=[
        pltpu.VMEM((TPW,), jnp.int32),
        pltpu.VMEM((TPW,), jnp.int32),
        pltpu.VMEM((TPW, D), jnp.float32),
        pltpu.VMEM((TPW, D), jnp.float32),
        pltpu.SemaphoreType.DMA,
        pltpu.SemaphoreType.DMA,
    ],
)
def _sc_combine(y_hbm, p1_hbm, p2_hbm, out_hbm, p1_v, p2_v, r1, r2, sem1, sem2):
    wid = lax.axis_index("s") * _NC + lax.axis_index("c")
    base = wid * TPW
    pltpu.sync_copy(p1_hbm.at[pl.ds(base, TPW)], p1_v)
    pltpu.sync_copy(p2_hbm.at[pl.ds(base, TPW)], p2_v)
    c1 = pltpu.async_copy(y_hbm.at[p1_v], r1, sem1)
    c2 = pltpu.async_copy(y_hbm.at[p2_v], r2, sem2)
    c1.wait()
    c2.wait()

    def tok_body(t, carry):
        def chunk_body(k, c):
            sl = pl.ds(k * 16, 16)
            r1[t, sl] = r1[t, sl] + r2[t, sl]
            return c
        return lax.fori_loop(0, NCH, chunk_body, carry)

    lax.fori_loop(0, TPW, tok_body, 0)
    pltpu.sync_copy(r1, out_hbm.at[pl.ds(base, TPW)])


@jax.jit
def _run(hs2d, gate_w, w_gate, w_up, w_down):
    logits = _logits(hs2d, gate_w)
    rw = jax.nn.softmax(logits, axis=1)
    rw_topk, sel = jax.lax.top_k(rw, TOPK)
    rwf = rw_topk.astype(jnp.float32)
    rwf = rwf / rwf.sum(axis=-1, keepdims=True)
    rwb = rwf.astype(jnp.bfloat16)

    eflat = jnp.concatenate([sel[:, 0], sel[:, 1]]).astype(jnp.int32)
    oh = (eflat[:, None] == jnp.arange(E, dtype=jnp.int32)[None, :]).astype(jnp.int32)
    cum = jnp.cumsum(oh, axis=0) - oh
    rank = jnp.take_along_axis(cum, eflat[:, None], axis=1)[:, 0]
    counts = jnp.sum(oh, axis=0)
    padded = ((counts + TTR - 1) // TTR) * TTR
    pcum = jnp.cumsum(padded)
    seg = pcum - padded
    pos = seg[eflat] + rank
    tok = jnp.concatenate([jnp.arange(S, dtype=jnp.int32)] * 2)
    srcrow = jnp.zeros((NPAD,), jnp.int32).at[pos].set(tok)
    wflat = jnp.concatenate([rwb[:, 0], rwb[:, 1]]).astype(jnp.float32)
    w_sorted = jnp.zeros((NPAD,), jnp.float32).at[pos].set(wflat)
    p1 = pos[:S]
    p2 = pos[S:]
    nact = (pcum[E - 1] // TTR).astype(jnp.int32)
    eot = jnp.searchsorted(pcum, jnp.arange(NTILES, dtype=jnp.int32) * TTR,
                           side="right").astype(jnp.int32)
    eot = jnp.minimum(eot, E - 1)
    meta = jnp.concatenate([eot, nact[None]])

    hs_i32 = jax.lax.bitcast_convert_type(hs2d.reshape(S, DI, 2), jnp.int32)
    x_i32 = _sc_dispatch(hs_i32, srcrow)
    x_sorted = jax.lax.bitcast_convert_type(x_i32, jnp.bfloat16).reshape(NPAD, D)
    y = _ffn(meta, x_sorted, w_sorted.reshape(NPAD, 1), w_gate, w_up, w_down)
    out32 = _sc_combine(y, p1, p2)
    return out32.astype(jnp.bfloat16), logits


def kernel(hidden_states, gate_w, w_gate, w_up, w_down):
    bsz, seq, d = hidden_states.shape
    hs2d = hidden_states.reshape(-1, d)
    out, logits = _run(hs2d, gate_w, w_gate, w_up, w_down)
    return out.reshape(bsz, seq, d), logits


# speedup vs baseline: 1.0448x; 1.0448x over previous
import functools

import jax
import jax.numpy as jnp
from jax import lax
from jax.experimental import pallas as pl
from jax.experimental.pallas import tpu as pltpu
from jax.experimental.pallas import tpu_sc as plsc

B, S, D = 1, 2048, 768
FFN = 3072
E = 8
TOPK = 2
NPAIR = S * TOPK
TTR = 256
NPAD = NPAIR + E * TTR
NTILES = NPAD // TTR
DI = D // 2

_sc_info = plsc.get_sparse_core_info()
_NC = _sc_info.num_cores
NW = _NC * _sc_info.num_subcores
RPW = NPAD // NW
TPW = S // NW
NCH = D // 16


def _logits_body(hs_ref, gate_ref, logits_ref):
    logits_f = jax.lax.dot_general(
        hs_ref[...], gate_ref[...], (((1,), (1,)), ((), ())),
        preferred_element_type=jnp.float32)
    logits_ref[...] = logits_f.astype(jnp.bfloat16)


def _logits(hs2d, gate_w):
    return pl.pallas_call(
        _logits_body,
        out_shape=jax.ShapeDtypeStruct((S, E), jnp.bfloat16),
    )(hs2d, gate_w)


@functools.partial(
    pl.kernel,
    mesh=plsc.VectorSubcoreMesh(core_axis_name="c", subcore_axis_name="s"),
    out_type=jax.ShapeDtypeStruct((NPAD, DI), jnp.int32),
    scratch_types=[
        pltpu.VMEM((RPW,), jnp.int32),
        pltpu.VMEM((4, RPW // 4, DI), jnp.int32),
        pltpu.SemaphoreType.DMA,
        pltpu.SemaphoreType.DMA,
        pltpu.SemaphoreType.DMA,
        pltpu.SemaphoreType.DMA,
        pltpu.SemaphoreType.DMA,
        pltpu.SemaphoreType.DMA,
        pltpu.SemaphoreType.DMA,
        pltpu.SemaphoreType.DMA,
    ],
)
def _sc_dispatch(hs_hbm, idx_hbm, out_hbm, idx_v, rows_v,
                 g0, g1, g2, g3, w0, w1, w2, w3):
    wid = lax.axis_index("s") * _NC + lax.axis_index("c")
    base = wid * RPW
    ch = RPW // 4
    gsems = (g0, g1, g2, g3)
    wsems = (w0, w1, w2, w3)
    pltpu.sync_copy(idx_hbm.at[pl.ds(base, RPW)], idx_v)
    gcopies = []
    for b in range(4):
        gcopies.append(pltpu.async_copy(
            hs_hbm.at[idx_v.at[pl.ds(b * ch, ch)]], rows_v.at[b], gsems[b]))
    wcopies = []
    for b in range(4):
        gcopies[b].wait()
        wcopies.append(pltpu.async_copy(
            rows_v.at[b], out_hbm.at[pl.ds(base + b * ch, ch)], wsems[b]))
    for b in range(4):
        wcopies[b].wait()


def _ffn_body(meta_ref, x_ref, w_ref, wg_ref, wu_ref, wd_ref, y_ref):
    j = pl.program_id(0)

    @pl.when(j < meta_ref[NTILES])
    def _compute():
        x = x_ref[...]
        g = jax.lax.dot_general(x, wg_ref[0], (((1,), (1,)), ((), ())),
                                preferred_element_type=jnp.float32)
        u = jax.lax.dot_general(x, wu_ref[0], (((1,), (1,)), ((), ())),
                                preferred_element_type=jnp.float32)
        h = (g * jax.nn.sigmoid(g) * u).astype(jnp.bfloat16)
        d = jax.lax.dot_general(h, wd_ref[0], (((1,), (1,)), ((), ())),
                                preferred_element_type=jnp.float32)
        y_ref[...] = d.astype(jnp.bfloat16).astype(jnp.float32) * w_ref[...]


def _ffn(meta, x_sorted, w_sorted, w_gate, w_up, w_down):
    grid_spec = pltpu.PrefetchScalarGridSpec(
        num_scalar_prefetch=1,
        grid=(NTILES,),
        in_specs=[
            pl.BlockSpec((TTR, D), lambda j, m: (j, 0)),
            pl.BlockSpec((TTR, 1), lambda j, m: (j, 0)),
            pl.BlockSpec((1, FFN, D), lambda j, m: (m[j], 0, 0)),
            pl.BlockSpec((1, FFN, D), lambda j, m: (m[j], 0, 0)),
            pl.BlockSpec((1, D, FFN), lambda j, m: (m[j], 0, 0)),
        ],
        out_specs=pl.BlockSpec((TTR, D), lambda j, m: (j, 0)),
    )
    return pl.pallas_call(
        _ffn_body,
        grid_spec=grid_spec,
        out_shape=jax.ShapeDtypeStruct((NPAD, D), jnp.float32),
    )(meta, x_sorted, w_sorted, w_gate, w_up, w_down)


@functools.partial(
    pl.kernel,
    mesh=plsc.VectorSubcoreMesh(core_axis_name="c", subcore_axis_name="s"),
    out_type=jax.ShapeDtypeStruct((S, D), jnp.float32),
    scratch_types=[
        pltpu.VMEM((TPW,), jnp.int32),
        pltpu.VMEM((TPW,), jnp.int32),
        pltpu.VMEM((TPW, D), jnp.float32),
        pltpu.VMEM((TPW, D), jnp.float32),
        pltpu.SemaphoreType.DMA,
        pltpu.SemaphoreType.DMA,
    ],
)
def _sc_combine(y_hbm, p1_hbm, p2_hbm, out_hbm, p1_v, p2_v, r1, r2, sem1, sem2):
    wid = lax.axis_index("s") * _NC + lax.axis_index("c")
    base = wid * TPW
    pltpu.sync_copy(p1_hbm.at[pl.ds(base, TPW)], p1_v)
    pltpu.sync_copy(p2_hbm.at[pl.ds(base, TPW)], p2_v)
    c1 = pltpu.async_copy(y_hbm.at[p1_v], r1, sem1)
    c2 = pltpu.async_copy(y_hbm.at[p2_v], r2, sem2)
    c1.wait()
    c2.wait()

    def tok_body(t, carry):
        def chunk_body(k, c):
            sl = pl.ds(k * 16, 16)
            r1[t, sl] = r1[t, sl] + r2[t, sl]
            return c
        return lax.fori_loop(0, NCH, chunk_body, carry)

    lax.fori_loop(0, TPW, tok_body, 0)
    pltpu.sync_copy(r1, out_hbm.at[pl.ds(base, TPW)])


@jax.jit
def _run(hs2d, gate_w, w_gate, w_up, w_down):
    logits = _logits(hs2d, gate_w)
    rw = jax.nn.softmax(logits, axis=1)
    rw_topk, sel = jax.lax.top_k(rw, TOPK)
    rwf = rw_topk.astype(jnp.float32)
    rwf = rwf / rwf.sum(axis=-1, keepdims=True)
    rwb = rwf.astype(jnp.bfloat16)

    eflat = jnp.concatenate([sel[:, 0], sel[:, 1]]).astype(jnp.int32)
    oh = (eflat[:, None] == jnp.arange(E, dtype=jnp.int32)[None, :]).astype(jnp.int32)
    cum = jnp.cumsum(oh, axis=0) - oh
    rank = jnp.take_along_axis(cum, eflat[:, None], axis=1)[:, 0]
    counts = jnp.sum(oh, axis=0)
    padded = ((counts + TTR - 1) // TTR) * TTR
    pcum = jnp.cumsum(padded)
    seg = pcum - padded
    pos = seg[eflat] + rank
    tok = jnp.concatenate([jnp.arange(S, dtype=jnp.int32)] * 2)
    srcrow = jnp.zeros((NPAD,), jnp.int32).at[pos].set(tok)
    wflat = jnp.concatenate([rwb[:, 0], rwb[:, 1]]).astype(jnp.float32)
    w_sorted = jnp.zeros((NPAD,), jnp.float32).at[pos].set(wflat)
    p1 = pos[:S]
    p2 = pos[S:]
    nact = (pcum[E - 1] // TTR).astype(jnp.int32)
    eot = jnp.searchsorted(pcum, jnp.arange(NTILES, dtype=jnp.int32) * TTR,
                           side="right").astype(jnp.int32)
    eot = jnp.minimum(eot, E - 1)
    meta = jnp.concatenate([eot, nact[None]])

    hs_i32 = jax.lax.bitcast_convert_type(hs2d.reshape(S, DI, 2), jnp.int32)
    x_i32 = _sc_dispatch(hs_i32, srcrow)
    x_sorted = jax.lax.bitcast_convert_type(x_i32, jnp.bfloat16).reshape(NPAD, D)
    y = _ffn(meta, x_sorted, w_sorted.reshape(NPAD, 1), w_gate, w_up, w_down)
    out32 = _sc_combine(y, p1, p2)
    return out32.astype(jnp.bfloat16), logits


def kernel(hidden_states, gate_w, w_gate, w_up, w_down):
    bsz, seq, d = hidden_states.shape
    hs2d = hidden_states.reshape(-1, d)
    out, logits = _run(hs2d, gate_w, w_gate, w_up, w_down)
    return out.reshape(bsz, seq, d), logits


# speedup vs baseline: 1.8831x; 1.8023x over previous
import functools

import jax
import jax.numpy as jnp
from jax import lax
from jax.experimental import pallas as pl
from jax.experimental.pallas import tpu as pltpu
from jax.experimental.pallas import tpu_sc as plsc

B, S, D = 1, 2048, 768
FFN = 3072
E = 8
TOPK = 2
NPAIR = S * TOPK
TTR = 256
NPAD = NPAIR + E * TTR
NTILES = NPAD // TTR
DI = D // 2

_sc_info = plsc.get_sparse_core_info()
_NC = _sc_info.num_cores
NW = _NC * _sc_info.num_subcores
RPW = NPAD // NW
TPW = S // NW
NCH = D // 16


def _logits_body(hs_ref, gate_ref, logits_ref):
    logits_f = jax.lax.dot_general(
        hs_ref[...], gate_ref[...], (((1,), (1,)), ((), ())),
        preferred_element_type=jnp.float32)
    logits_ref[...] = logits_f.astype(jnp.bfloat16)


def _logits(hs2d, gate_w):
    return pl.pallas_call(
        _logits_body,
        out_shape=jax.ShapeDtypeStruct((S, E), jnp.bfloat16),
    )(hs2d, gate_w)


def _ffn_body(meta_ref, sr_ref, w_ref, hs_ref, wg_ref, wu_ref, wd_ref, y_ref):
    j = pl.program_id(0)

    @pl.when(j < meta_ref[NTILES])
    def _compute():
        sr = sr_ref[...]
        tok_iota = jax.lax.broadcasted_iota(jnp.int32, (TTR, S), 1)
        p1h = jnp.where(tok_iota == sr, 1.0, 0.0).astype(jnp.bfloat16)
        x = jax.lax.dot_general(p1h, hs_ref[...], (((1,), (0,)), ((), ())),
                                preferred_element_type=jnp.float32
                                ).astype(jnp.bfloat16)
        g = jax.lax.dot_general(x, wg_ref[0], (((1,), (1,)), ((), ())),
                                preferred_element_type=jnp.float32)
        u = jax.lax.dot_general(x, wu_ref[0], (((1,), (1,)), ((), ())),
                                preferred_element_type=jnp.float32)
        h = (g * jax.nn.sigmoid(g) * u).astype(jnp.bfloat16)
        d = jax.lax.dot_general(h, wd_ref[0], (((1,), (1,)), ((), ())),
                                preferred_element_type=jnp.float32)
        y_ref[...] = d.astype(jnp.bfloat16).astype(jnp.float32) * w_ref[...]


def _ffn(meta, srcrow, w_sorted, hs2d, w_gate, w_up, w_down):
    grid_spec = pltpu.PrefetchScalarGridSpec(
        num_scalar_prefetch=1,
        grid=(NTILES,),
        in_specs=[
            pl.BlockSpec((TTR, 1), lambda j, m: (j, 0)),
            pl.BlockSpec((TTR, 1), lambda j, m: (j, 0)),
            pl.BlockSpec((S, D), lambda j, m: (0, 0)),
            pl.BlockSpec((1, FFN, D), lambda j, m: (m[j], 0, 0)),
            pl.BlockSpec((1, FFN, D), lambda j, m: (m[j], 0, 0)),
            pl.BlockSpec((1, D, FFN), lambda j, m: (m[j], 0, 0)),
        ],
        out_specs=pl.BlockSpec((TTR, D), lambda j, m: (j, 0)),
    )
    return pl.pallas_call(
        _ffn_body,
        grid_spec=grid_spec,
        out_shape=jax.ShapeDtypeStruct((NPAD, D), jnp.float32),
    )(meta, srcrow, w_sorted, hs2d, w_gate, w_up, w_down)


@functools.partial(
    pl.kernel,
    mesh=plsc.VectorSubcoreMesh(core_axis_name="c", subcore_axis_name="s"),
    out_type=jax.ShapeDtypeStruct((S, D), jnp.float32),
    scratch_types=[
        pltpu.VMEM((TPW,), jnp.int32),
        pltpu.VMEM((TPW,), jnp.int32),
        pltpu.VMEM((TPW, D), jnp.float32),
        pltpu.VMEM((TPW, D), jnp.float32),
        pltpu.SemaphoreType.DMA,
        pltpu.SemaphoreType.DMA,
    ],
)
def _sc_combine(y_hbm, p1_hbm, p2_hbm, out_hbm, p1_v, p2_v, r1, r2, sem1, sem2):
    wid = lax.axis_index("s") * _NC + lax.axis_index("c")
    base = wid * TPW
    pltpu.sync_copy(p1_hbm.at[pl.ds(base, TPW)], p1_v)
    pltpu.sync_copy(p2_hbm.at[pl.ds(base, TPW)], p2_v)
    c1 = pltpu.async_copy(y_hbm.at[p1_v], r1, sem1)
    c2 = pltpu.async_copy(y_hbm.at[p2_v], r2, sem2)
    c1.wait()
    c2.wait()

    def tok_body(t, carry):
        def chunk_body(k, c):
            sl = pl.ds(k * 16, 16)
            r1[t, sl] = r1[t, sl] + r2[t, sl]
            return c
        return lax.fori_loop(0, NCH, chunk_body, carry)

    lax.fori_loop(0, TPW, tok_body, 0)
    pltpu.sync_copy(r1, out_hbm.at[pl.ds(base, TPW)])


@jax.jit
def _run(hs2d, gate_w, w_gate, w_up, w_down):
    logits = _logits(hs2d, gate_w)
    rw = jax.nn.softmax(logits, axis=1)
    rw_topk, sel = jax.lax.top_k(rw, TOPK)
    rwf = rw_topk.astype(jnp.float32)
    rwf = rwf / rwf.sum(axis=-1, keepdims=True)
    rwb = rwf.astype(jnp.bfloat16)

    eflat = jnp.concatenate([sel[:, 0], sel[:, 1]]).astype(jnp.int32)
    oh = (eflat[:, None] == jnp.arange(E, dtype=jnp.int32)[None, :]).astype(jnp.int32)
    cum = jnp.cumsum(oh, axis=0) - oh
    rank = jnp.take_along_axis(cum, eflat[:, None], axis=1)[:, 0]
    counts = jnp.sum(oh, axis=0)
    padded = ((counts + TTR - 1) // TTR) * TTR
    pcum = jnp.cumsum(padded)
    seg = pcum - padded
    pos = seg[eflat] + rank
    tok = jnp.concatenate([jnp.arange(S, dtype=jnp.int32)] * 2)
    srcrow = jnp.zeros((NPAD,), jnp.int32).at[pos].set(tok)
    wflat = jnp.concatenate([rwb[:, 0], rwb[:, 1]]).astype(jnp.float32)
    w_sorted = jnp.zeros((NPAD,), jnp.float32).at[pos].set(wflat)
    p1 = pos[:S]
    p2 = pos[S:]
    nact = (pcum[E - 1] // TTR).astype(jnp.int32)
    eot = jnp.searchsorted(pcum, jnp.arange(NTILES, dtype=jnp.int32) * TTR,
                           side="right").astype(jnp.int32)
    eot = jnp.minimum(eot, E - 1)
    meta = jnp.concatenate([eot, nact[None]])

    y = _ffn(meta, srcrow.reshape(NPAD, 1), w_sorted.reshape(NPAD, 1),
             hs2d, w_gate, w_up, w_down)
    out32 = _sc_combine(y, p1, p2)
    return out32.astype(jnp.bfloat16), logits


def kernel(hidden_states, gate_w, w_gate, w_up, w_down):
    bsz, seq, d = hidden_states.shape
    hs2d = hidden_states.reshape(-1, d)
    out, logits = _run(hs2d, gate_w, w_gate, w_up, w_down)
    return out.reshape(bsz, seq, d), logits


# speedup vs baseline: 2.0365x; 1.0814x over previous
import functools

import jax
import jax.numpy as jnp
from jax import lax
from jax.experimental import pallas as pl
from jax.experimental.pallas import tpu as pltpu
from jax.experimental.pallas import tpu_sc as plsc

B, S, D = 1, 2048, 768
FFN = 3072
E = 8
TOPK = 2
NPAIR = S * TOPK
TTR = 256
NPAD = NPAIR + E * TTR
NTILES = NPAD // TTR
DI = D // 2

_sc_info = plsc.get_sparse_core_info()
_NC = _sc_info.num_cores
NW = _NC * _sc_info.num_subcores
RPW = NPAD // NW
TPW = S // NW
NCH = D // 16


def _logits_body(hs_ref, gate_ref, logits_ref):
    logits_f = jax.lax.dot_general(
        hs_ref[...], gate_ref[...], (((1,), (1,)), ((), ())),
        preferred_element_type=jnp.float32)
    logits_ref[...] = logits_f.astype(jnp.bfloat16)


def _logits(hs2d, gate_w):
    return pl.pallas_call(
        _logits_body,
        out_shape=jax.ShapeDtypeStruct((S, E), jnp.bfloat16),
    )(hs2d, gate_w)


def _ffn_body(meta_ref, pk_ref, hs_ref, wg_ref, wu_ref, wd_ref, y_ref):
    j = pl.program_id(0)

    @pl.when(j < meta_ref[NTILES])
    def _compute():
        pk = pk_ref[...]
        sr = jnp.bitwise_and(pk, 0xFFFF)
        w = jax.lax.bitcast_convert_type(
            jnp.bitwise_and(pk, jnp.int32(-65536)), jnp.float32)
        tok_iota = jax.lax.broadcasted_iota(jnp.int32, (TTR, S), 1)
        p1h = jnp.where(tok_iota == sr, 1.0, 0.0).astype(jnp.bfloat16)
        x = jax.lax.dot_general(p1h, hs_ref[...], (((1,), (0,)), ((), ())),
                                preferred_element_type=jnp.float32
                                ).astype(jnp.bfloat16)
        g = jax.lax.dot_general(x, wg_ref[0], (((1,), (1,)), ((), ())),
                                preferred_element_type=jnp.float32)
        u = jax.lax.dot_general(x, wu_ref[0], (((1,), (1,)), ((), ())),
                                preferred_element_type=jnp.float32)
        h = (g * jax.nn.sigmoid(g) * u).astype(jnp.bfloat16)
        d = jax.lax.dot_general(h, wd_ref[0], (((1,), (1,)), ((), ())),
                                preferred_element_type=jnp.float32)
        y_ref[...] = d.astype(jnp.bfloat16).astype(jnp.float32) * w


def _ffn(meta, sortpk, hs2d, w_gate, w_up, w_down):
    grid_spec = pltpu.PrefetchScalarGridSpec(
        num_scalar_prefetch=1,
        grid=(NTILES,),
        in_specs=[
            pl.BlockSpec((TTR, 1), lambda j, m: (j, 0)),
            pl.BlockSpec((S, D), lambda j, m: (0, 0)),
            pl.BlockSpec((1, FFN, D), lambda j, m: (m[j], 0, 0)),
            pl.BlockSpec((1, FFN, D), lambda j, m: (m[j], 0, 0)),
            pl.BlockSpec((1, D, FFN), lambda j, m: (m[j], 0, 0)),
        ],
        out_specs=pl.BlockSpec((TTR, D), lambda j, m: (j, 0)),
    )
    return pl.pallas_call(
        _ffn_body,
        grid_spec=grid_spec,
        out_shape=jax.ShapeDtypeStruct((NPAD, D), jnp.float32),
    )(meta, sortpk, hs2d, w_gate, w_up, w_down)


@functools.partial(
    pl.kernel,
    mesh=plsc.VectorSubcoreMesh(core_axis_name="c", subcore_axis_name="s"),
    out_type=jax.ShapeDtypeStruct((S, D), jnp.float32),
    scratch_types=[
        pltpu.VMEM((TPW,), jnp.int32),
        pltpu.VMEM((TPW,), jnp.int32),
        pltpu.VMEM((TPW, D), jnp.float32),
        pltpu.VMEM((TPW, D), jnp.float32),
        pltpu.SemaphoreType.DMA,
        pltpu.SemaphoreType.DMA,
    ],
)
def _sc_combine(y_hbm, p1_hbm, p2_hbm, out_hbm, p1_v, p2_v, r1, r2, sem1, sem2):
    wid = lax.axis_index("s") * _NC + lax.axis_index("c")
    base = wid * TPW
    pltpu.sync_copy(p1_hbm.at[pl.ds(base, TPW)], p1_v)
    pltpu.sync_copy(p2_hbm.at[pl.ds(base, TPW)], p2_v)
    c1 = pltpu.async_copy(y_hbm.at[p1_v], r1, sem1)
    c2 = pltpu.async_copy(y_hbm.at[p2_v], r2, sem2)
    c1.wait()
    c2.wait()

    def tok_body(t, carry):
        def chunk_body(k, c):
            sl = pl.ds(k * 16, 16)
            r1[t, sl] = r1[t, sl] + r2[t, sl]
            return c
        return lax.fori_loop(0, NCH, chunk_body, carry)

    lax.fori_loop(0, TPW, tok_body, 0)
    pltpu.sync_copy(r1, out_hbm.at[pl.ds(base, TPW)])


@jax.jit
def _run(hs2d, gate_w, w_gate, w_up, w_down):
    logits = _logits(hs2d, gate_w)
    rw = jax.nn.softmax(logits, axis=1)
    rw_topk, sel = jax.lax.top_k(rw, TOPK)
    rwf = rw_topk.astype(jnp.float32)
    rwf = rwf / rwf.sum(axis=-1, keepdims=True)
    rwb = rwf.astype(jnp.bfloat16)

    eflat = jnp.concatenate([sel[:, 0], sel[:, 1]]).astype(jnp.int32)
    oh = (eflat[:, None] == jnp.arange(E, dtype=jnp.int32)[None, :]).astype(jnp.int32)
    cum = jnp.cumsum(oh, axis=0) - oh
    rank = jnp.take_along_axis(cum, eflat[:, None], axis=1)[:, 0]
    counts = jnp.sum(oh, axis=0)
    padded = ((counts + TTR - 1) // TTR) * TTR
    pcum = jnp.cumsum(padded)
    seg = pcum - padded
    pos = seg[eflat] + rank
    tok = jnp.concatenate([jnp.arange(S, dtype=jnp.int32)] * 2)
    wbits = jax.lax.bitcast_convert_type(
        jnp.concatenate([rwb[:, 0], rwb[:, 1]]), jnp.uint16).astype(jnp.int32)
    packed = jnp.bitwise_or(jnp.left_shift(wbits, 16), tok)
    sortpk = jnp.zeros((NPAD,), jnp.int32).at[pos].set(
        packed, unique_indices=True, mode="promise_in_bounds")
    p1 = pos[:S]
    p2 = pos[S:]
    nact = (pcum[E - 1] // TTR).astype(jnp.int32)
    eot = jnp.searchsorted(pcum, jnp.arange(NTILES, dtype=jnp.int32) * TTR,
                           side="right").astype(jnp.int32)
    eot = jnp.minimum(eot, E - 1)
    meta = jnp.concatenate([eot, nact[None]])

    y = _ffn(meta, sortpk.reshape(NPAD, 1), hs2d, w_gate, w_up, w_down)
    out32 = _sc_combine(y, p1, p2)
    return out32.astype(jnp.bfloat16), logits


def kernel(hidden_states, gate_w, w_gate, w_up, w_down):
    bsz, seq, d = hidden_states.shape
    hs2d = hidden_states.reshape(-1, d)
    out, logits = _run(hs2d, gate_w, w_gate, w_up, w_down)
    return out.reshape(bsz, seq, d), logits
